# baseline (speedup 1.0000x reference)
import functools

import jax
import jax.numpy as jnp
from jax import lax
from jax.experimental import pallas as pl
from jax.experimental.pallas import tpu as pltpu
from jax.experimental.pallas import tpu_sc as plsc

INPUT_DIM = 768
OUTPUT_DIM = 768
NUM_EXPERTS = 16
TOP_K = 2
BATCH = 64

NW = 32
I_PER = INPUT_DIM // NW
L = 16
PK = BATCH * TOP_K
CCH = OUTPUT_DIM // L
OCH = 128
NOCH = OUTPUT_DIM // OCH
BIAS_W = BATCH // 8


def _iota():
    return lax.broadcasted_iota(jnp.int32, (L,), 0)


def _splat(x):
    return jnp.full((L,), x, jnp.int32)


def _sc_body(wp_hbm, wi_hbm, bp_hbm, bi_hbm, wb_hbm, bb_hbm,
             outw_hbm, outb_hbm,
             bank_v, wp_v, wi_v, bp_c, bi_c, bbank_c,
             outw_v, outb_v, sem):
    cid = lax.axis_index("c")
    sid = lax.axis_index("s")
    wid = sid * 2 + cid
    iov = _iota()

    @pl.when(wid < BIAS_W)
    def _bias():
        b0 = wid * 8
        for ch in range(NOCH):
            pltpu.sync_copy(bp_hbm.at[pl.ds(ch * OCH * PK, OCH * PK)], bp_c)
            pltpu.sync_copy(bi_hbm.at[pl.ds(ch * OCH * PK, OCH * PK)], bi_c)
            pltpu.sync_copy(
                bb_hbm.at[pl.ds(ch * OCH * NUM_EXPERTS, OCH * NUM_EXPERTS)],
                bbank_c)
            for oc in range(OCH // L):
                ol = iov + (oc * L)
                olp = ol * PK
                for t in range(8):
                    bsp = _splat((b0 + t) * 2)
                    p0 = plsc.load_gather(bp_c, [olp + bsp])
                    p1 = plsc.load_gather(bp_c, [olp + bsp + 1])
                    e0 = plsc.load_gather(bi_c, [olp + bsp])
                    e1 = plsc.load_gather(bi_c, [olp + bsp + 1])
                    v0 = plsc.load_gather(bbank_c, [ol * NUM_EXPERTS + e0])
                    v1 = plsc.load_gather(bbank_c, [ol * NUM_EXPERTS + e1])
                    plsc.store_scatter(
                        outb_v,
                        [_splat(t * OUTPUT_DIM + ch * OCH + oc * L) + iov],
                        p0 * v0 + p1 * v1)
        pltpu.sync_copy(outb_v,
                        outb_hbm.at[pl.ds(b0 * OUTPUT_DIM, 8 * OUTPUT_DIM)])

    i0 = wid * I_PER
    pltpu.sync_copy(wp_hbm.at[pl.ds(i0 * PK, I_PER * PK)], wp_v)
    pltpu.sync_copy(wi_hbm.at[pl.ds(i0 * PK, I_PER * PK)], wi_v)

    def i_body(il, _):
        i = i0 + il
        pltpu.sync_copy(
            wb_hbm.at[pl.ds(i * NUM_EXPERTS * OUTPUT_DIM,
                            NUM_EXPERTS * OUTPUT_DIM)], bank_v)
        ilp = _splat(il * PK)

        def t_body(b, _):
            bsp = ilp + 2 * b
            p0 = plsc.load_gather(wp_v, [bsp])
            p1 = plsc.load_gather(wp_v, [bsp + 1])
            e0 = plsc.load_gather(wi_v, [bsp]) * OUTPUT_DIM
            e1 = plsc.load_gather(wi_v, [bsp + 1]) * OUTPUT_DIM
            for c in range(CCH):
                col = iov + (c * L)
                v0 = plsc.load_gather(bank_v, [e0 + col])
                v1 = plsc.load_gather(bank_v, [e1 + col])
                outw_v[pl.ds(c * L, L)] = p0 * v0 + p1 * v1
            pltpu.async_copy(
                outw_v,
                outw_hbm.at[pl.ds((b * INPUT_DIM + i) * OUTPUT_DIM,
                                  OUTPUT_DIM)],
                sem).wait()
            return 0

        lax.fori_loop(0, BATCH, t_body, 0)
        return 0

    lax.fori_loop(0, I_PER, i_body, 0)


@jax.jit
def kernel(weight_probs, weight_indices, bias_probs, bias_indices,
           weight_bank, bias_bank):
    wp = weight_probs.reshape(-1)
    wi = weight_indices.reshape(-1)
    bp = bias_probs.reshape(-1)
    bi = bias_indices.reshape(-1)
    wb = weight_bank.reshape(-1)
    bb = bias_bank.reshape(-1)

    mesh = plsc.VectorSubcoreMesh(core_axis_name="c", subcore_axis_name="s")
    outw, outb = pl.kernel(
        _sc_body,
        out_type=(
            jax.ShapeDtypeStruct((BATCH * INPUT_DIM * OUTPUT_DIM,),
                                 jnp.float32),
            jax.ShapeDtypeStruct((BATCH * OUTPUT_DIM,), jnp.float32),
        ),
        mesh=mesh,
        compiler_params=pltpu.CompilerParams(needs_layout_passes=False),
        scratch_types=(
            pltpu.VMEM((NUM_EXPERTS * OUTPUT_DIM,), jnp.float32),
            pltpu.VMEM((I_PER * PK,), jnp.float32),
            pltpu.VMEM((I_PER * PK,), jnp.int32),
            pltpu.VMEM((OCH * PK,), jnp.float32),
            pltpu.VMEM((OCH * PK,), jnp.int32),
            pltpu.VMEM((OCH * NUM_EXPERTS,), jnp.float32),
            pltpu.VMEM((OUTPUT_DIM,), jnp.float32),
            pltpu.VMEM((8 * OUTPUT_DIM,), jnp.float32),
            pltpu.SemaphoreType.DMA,
        ),
    )(wp, wi, bp, bi, wb, bb)
    return (outw.reshape(BATCH, INPUT_DIM, OUTPUT_DIM),
            outb.reshape(BATCH, OUTPUT_DIM))


# speedup vs baseline: 1.1676x; 1.1676x over previous
import functools

import jax
import jax.numpy as jnp
from jax import lax
from jax.experimental import pallas as pl
from jax.experimental.pallas import tpu as pltpu
from jax.experimental.pallas import tpu_sc as plsc

INPUT_DIM = 768
OUTPUT_DIM = 768
NUM_EXPERTS = 16
TOP_K = 2
BATCH = 64

NW = 32
I_PER = INPUT_DIM // NW
L = 16
PK = BATCH * TOP_K
CCH = OUTPUT_DIM // L
BANK_W = NUM_EXPERTS * OUTPUT_DIM
OCH = 32
NOCH = OUTPUT_DIM // OCH
BIAS_W = BATCH // 8


def _iota():
    return lax.broadcasted_iota(jnp.int32, (L,), 0)


def _splat(x):
    return jnp.full((L,), x, jnp.int32)


def _sc_body(wp_hbm, wi_hbm, bp_hbm, bi_hbm, wb_hbm, bb_hbm,
             outw_hbm, outb_hbm,
             bank_a, bank_b, wp_v, wi_v,
             bpc_a, bpc_b, bic_a, bic_b, bbk_a, bbk_b,
             outw_v, outb_v,
             sem_ba, sem_bb, sem_out, sem_bias):
    cid = lax.axis_index("c")
    sid = lax.axis_index("s")
    wid = sid * 2 + cid
    iov = _iota()

    def bank_src(i):
        return wb_hbm.at[pl.ds(i * BANK_W, BANK_W)]

    @pl.when(wid < BIAS_W)
    def _bias():
        b0 = wid * 8

        def fire_bias(ch, bufs):
            off = ch * OCH
            pltpu.async_copy(bp_hbm.at[pl.ds(off * PK, OCH * PK)],
                             bufs[0], sem_bias)
            pltpu.async_copy(bi_hbm.at[pl.ds(off * PK, OCH * PK)],
                             bufs[1], sem_bias)
            pltpu.async_copy(
                bb_hbm.at[pl.ds(off * NUM_EXPERTS, OCH * NUM_EXPERTS)],
                bufs[2], sem_bias)

        def wait_bias(bufs):
            pltpu.make_async_copy(bp_hbm.at[pl.ds(0, OCH * PK)],
                                  bufs[0], sem_bias).wait()
            pltpu.make_async_copy(bi_hbm.at[pl.ds(0, OCH * PK)],
                                  bufs[1], sem_bias).wait()
            pltpu.make_async_copy(
                bb_hbm.at[pl.ds(0, OCH * NUM_EXPERTS)], bufs[2],
                sem_bias).wait()

        bufs = [(bpc_a, bic_a, bbk_a), (bpc_b, bic_b, bbk_b)]
        fire_bias(0, bufs[0])
        for ch in range(NOCH):
            cur = bufs[ch % 2]
            wait_bias(cur)
            if ch + 1 < NOCH:
                fire_bias(ch + 1, bufs[(ch + 1) % 2])
            bp_c, bi_c, bbank_c = cur
            for oc in range(OCH // L):
                olp = (iov + oc * L) * PK
                ole = (iov + oc * L) * NUM_EXPERTS
                for t in range(8):
                    bsp = _splat((b0 + t) * 2)
                    p0 = plsc.load_gather(bp_c, [olp + bsp])
                    p1 = plsc.load_gather(bp_c, [olp + bsp + 1])
                    e0 = plsc.load_gather(bi_c, [olp + bsp])
                    e1 = plsc.load_gather(bi_c, [olp + bsp + 1])
                    v0 = plsc.load_gather(bbank_c, [ole + e0])
                    v1 = plsc.load_gather(bbank_c, [ole + e1])
                    plsc.store_scatter(
                        outb_v,
                        [_splat(t * OUTPUT_DIM + ch * OCH + oc * L) + iov],
                        p0 * v0 + p1 * v1)
        pltpu.sync_copy(outb_v,
                        outb_hbm.at[pl.ds(b0 * OUTPUT_DIM, 8 * OUTPUT_DIM)])

    i0 = wid * I_PER
    pltpu.sync_copy(wp_hbm.at[pl.ds(i0 * PK, I_PER * PK)], wp_v)
    pltpu.sync_copy(wi_hbm.at[pl.ds(i0 * PK, I_PER * PK)], wi_v)

    pltpu.async_copy(bank_src(i0), bank_a, sem_ba)

    def compute_row(i, il, bank_v):
        ilp = _splat(il * PK)

        def t_body(b, _):
            bsp = ilp + 2 * b
            p0 = plsc.load_gather(wp_v, [bsp])
            p1 = plsc.load_gather(wp_v, [bsp + 1])
            ec0 = plsc.load_gather(wi_v, [bsp]) * OUTPUT_DIM + iov
            ec1 = plsc.load_gather(wi_v, [bsp + 1]) * OUTPUT_DIM + iov
            ob = _splat(b * OUTPUT_DIM) + iov
            for c in range(CCH):
                v0 = plsc.load_gather(bank_v, [ec0 + c * L])
                v1 = plsc.load_gather(bank_v, [ec1 + c * L])
                plsc.store_scatter(outw_v, [ob + c * L], p0 * v0 + p1 * v1)
            pltpu.async_copy(
                outw_v.at[pl.ds(b * OUTPUT_DIM, OUTPUT_DIM)],
                outw_hbm.at[pl.ds((b * INPUT_DIM + i) * OUTPUT_DIM,
                                  OUTPUT_DIM)],
                sem_out)
            return 0

        lax.fori_loop(0, BATCH, t_body, 0)
        pltpu.make_async_copy(
            outw_v, outw_hbm.at[pl.ds(0, BATCH * OUTPUT_DIM)],
            sem_out).wait()

    def pair_body(p, _):
        i_even = i0 + 2 * p
        pltpu.make_async_copy(bank_src(0), bank_a, sem_ba).wait()
        pltpu.async_copy(bank_src(i_even + 1), bank_b, sem_bb)
        compute_row(i_even, 2 * p, bank_a)
        pltpu.make_async_copy(bank_src(0), bank_b, sem_bb).wait()
        nxt = jnp.minimum(i_even + 2, INPUT_DIM - 1)
        pltpu.async_copy(bank_src(nxt), bank_a, sem_ba)
        compute_row(i_even + 1, 2 * p + 1, bank_b)
        return 0

    lax.fori_loop(0, I_PER // 2, pair_body, 0)
    pltpu.make_async_copy(bank_src(0), bank_a, sem_ba).wait()


@jax.jit
def kernel(weight_probs, weight_indices, bias_probs, bias_indices,
           weight_bank, bias_bank):
    wp = weight_probs.reshape(-1)
    wi = weight_indices.reshape(-1)
    bp = bias_probs.reshape(-1)
    bi = bias_indices.reshape(-1)
    wb = weight_bank.reshape(-1)
    bb = bias_bank.reshape(-1)

    mesh = plsc.VectorSubcoreMesh(core_axis_name="c", subcore_axis_name="s")
    outw, outb = pl.kernel(
        _sc_body,
        out_type=(
            jax.ShapeDtypeStruct((BATCH * INPUT_DIM * OUTPUT_DIM,),
                                 jnp.float32),
            jax.ShapeDtypeStruct((BATCH * OUTPUT_DIM,), jnp.float32),
        ),
        mesh=mesh,
        compiler_params=pltpu.CompilerParams(needs_layout_passes=False),
        scratch_types=(
            pltpu.VMEM((BANK_W,), jnp.float32),
            pltpu.VMEM((BANK_W,), jnp.float32),
            pltpu.VMEM((I_PER * PK,), jnp.float32),
            pltpu.VMEM((I_PER * PK,), jnp.int32),
            pltpu.VMEM((OCH * PK,), jnp.float32),
            pltpu.VMEM((OCH * PK,), jnp.float32),
            pltpu.VMEM((OCH * PK,), jnp.int32),
            pltpu.VMEM((OCH * PK,), jnp.int32),
            pltpu.VMEM((OCH * NUM_EXPERTS,), jnp.float32),
            pltpu.VMEM((OCH * NUM_EXPERTS,), jnp.float32),
            pltpu.VMEM((BATCH * OUTPUT_DIM,), jnp.float32),
            pltpu.VMEM((8 * OUTPUT_DIM,), jnp.float32),
            pltpu.SemaphoreType.DMA,
            pltpu.SemaphoreType.DMA,
            pltpu.SemaphoreType.DMA,
            pltpu.SemaphoreType.DMA,
        ),
    )(wp, wi, bp, bi, wb, bb)
    return (outw.reshape(BATCH, INPUT_DIM, OUTPUT_DIM),
            outb.reshape(BATCH, OUTPUT_DIM))
